# Initial kernel scaffold; baseline (speedup 1.0000x reference)
import jax
import jax.numpy as jnp
from jax.experimental import pallas as pl


def kernel(x, token_table, pos_table):
    raise NotImplementedError("write your pallas kernel here")


# speedup vs baseline: 1.1257x; 1.1257x over previous
import functools

import jax
import jax.numpy as jnp
from jax import lax
from jax.experimental import pallas as pl
from jax.experimental.pallas import tpu as pltpu
from jax.experimental.pallas import tpu_sc as plsc

VOCAB = 1000000
MAXLEN = 200
EMBED = 32
BATCH = 4096

NC = 2
NS = 16
NW = NC * NS
L = 16

SEQ_PER_W = BATCH // NW
NCH = 2
CH = MAXLEN // NCH


def _body(x_hbm, tok_hbm, pos_hbm, out_hbm, idx_v, rows_v, pos_v, sem):
    wid = lax.axis_index("s") * NC + lax.axis_index("c")

    pltpu.sync_copy(pos_hbm, pos_v)

    def seq_body(i, carry):
        seq = wid * SEQ_PER_W + i
        pltpu.sync_copy(x_hbm.at[seq], idx_v)
        cps = [
            pltpu.async_copy(tok_hbm.at[idx_v.at[j]], rows_v.at[j], sem)
            for j in range(NCH)
        ]
        for cp in cps:
            cp.wait()

        def add_body(r, c):
            for j in range(NCH):
                for h in range(EMBED // L):
                    sl = pl.ds(h * L, L)
                    rows_v[j, r, sl] = rows_v[j, r, sl] + pos_v[j, r, sl]
            return c

        lax.fori_loop(0, CH, add_body, 0)
        pltpu.sync_copy(rows_v, out_hbm.at[seq])
        return carry

    lax.fori_loop(0, SEQ_PER_W, seq_body, 0)


@jax.jit
def kernel(x, token_table, pos_table):
    x3 = x.reshape(BATCH, NCH, CH).astype(jnp.int32)
    pos3 = pos_table.reshape(NCH, CH, EMBED)
    mesh = plsc.VectorSubcoreMesh(
        core_axis_name="c", subcore_axis_name="s", num_cores=NC, num_subcores=NS
    )
    run = pl.kernel(
        _body,
        out_type=jax.ShapeDtypeStruct((BATCH, NCH, CH, EMBED), jnp.float32),
        mesh=mesh,
        scratch_types=[
            pltpu.VMEM((NCH, CH), jnp.int32),
            pltpu.VMEM((NCH, CH, EMBED), jnp.float32),
            pltpu.VMEM((NCH, CH, EMBED), jnp.float32),
            pltpu.SemaphoreType.DMA,
        ],
        compiler_params=pltpu.CompilerParams(use_tc_tiling_on_sc=False),
    )
    out = run(x3, token_table, pos3)
    return out.reshape(BATCH, MAXLEN, EMBED)


# speedup vs baseline: 1.3366x; 1.1874x over previous
import functools

import jax
import jax.numpy as jnp
from jax import lax
from jax.experimental import pallas as pl
from jax.experimental.pallas import tpu as pltpu
from jax.experimental.pallas import tpu_sc as plsc

VOCAB = 1000000
MAXLEN = 200
EMBED = 32
BATCH = 4096

NC = 2
NS = 16
NW = NC * NS
L = 16

SEQ_PER_W = BATCH // NW
NCH = 2
CH = MAXLEN // NCH
NB = 4
NT = SEQ_PER_W // NB


def _body(x_hbm, tok_hbm, pos_hbm, out_hbm, idx_v, pos_v, gbuf, obuf,
          gsems, osems):
    wid = lax.axis_index("s") * NC + lax.axis_index("c")

    pltpu.sync_copy(x_hbm.at[wid], idx_v)
    pltpu.sync_copy(pos_hbm, pos_v)

    def issue_gather(k, b):
        for j in range(NCH):
            pltpu.async_copy(
                tok_hbm.at[idx_v.at[k].at[j]], gbuf.at[b].at[j], gsems[b]
            )

    def wait_gather(k, b):
        for j in range(NCH):
            pltpu.make_async_copy(
                tok_hbm.at[idx_v.at[k].at[j]], gbuf.at[b].at[j], gsems[b]
            ).wait()

    for b in range(NB):
        issue_gather(b, b)

    def outer(t, carry):
        for b in range(NB):
            k = t * NB + b
            wait_gather(k, b)

            @pl.when(t > 0)
            def _():
                pltpu.make_async_copy(
                    obuf.at[b], out_hbm.at[wid * SEQ_PER_W + k - NB], osems[b]
                ).wait()

            def add_body(r, c):
                for j in range(NCH):
                    for h in range(EMBED // L):
                        sl = pl.ds(h * L, L)
                        obuf[b, j, r, sl] = gbuf[b, j, r, sl] + pos_v[j, r, sl]
                return c

            lax.fori_loop(0, CH, add_body, 0)
            pltpu.async_copy(obuf.at[b], out_hbm.at[wid * SEQ_PER_W + k],
                             osems[b])

            @pl.when(t < NT - 1)
            def _():
                issue_gather(k + NB, b)
        return carry

    lax.fori_loop(0, NT, outer, 0)

    for b in range(NB):
        pltpu.make_async_copy(
            obuf.at[b], out_hbm.at[wid * SEQ_PER_W + (NT - 1) * NB + b],
            osems[b]
        ).wait()


@jax.jit
def kernel(x, token_table, pos_table):
    x4 = x.reshape(NW, SEQ_PER_W, NCH, CH).astype(jnp.int32)
    pos3 = pos_table.reshape(NCH, CH, EMBED)
    mesh = plsc.VectorSubcoreMesh(
        core_axis_name="c", subcore_axis_name="s", num_cores=NC, num_subcores=NS
    )
    run = pl.kernel(
        _body,
        out_type=jax.ShapeDtypeStruct((BATCH, NCH, CH, EMBED), jnp.float32),
        mesh=mesh,
        scratch_types=[
            pltpu.VMEM((SEQ_PER_W, NCH, CH), jnp.int32),
            pltpu.VMEM((NCH, CH, EMBED), jnp.float32),
            pltpu.VMEM((NB, NCH, CH, EMBED), jnp.float32),
            pltpu.VMEM((NB, NCH, CH, EMBED), jnp.float32),
            [pltpu.SemaphoreType.DMA] * NB,
            [pltpu.SemaphoreType.DMA] * NB,
        ],
        compiler_params=pltpu.CompilerParams(use_tc_tiling_on_sc=False),
    )
    out = run(x4, token_table, pos3)
    return out.reshape(BATCH, MAXLEN, EMBED)


# speedup vs baseline: 1.4900x; 1.1147x over previous
import functools

import jax
import jax.numpy as jnp
from jax import lax
from jax.experimental import pallas as pl
from jax.experimental.pallas import tpu as pltpu
from jax.experimental.pallas import tpu_sc as plsc

VOCAB = 1000000
MAXLEN = 200
EMBED = 32
BATCH = 4096

NC = 2
NS = 16
NW = NC * NS
L = 16

SEQ_PER_W = BATCH // NW
NCH = 2
CH = MAXLEN // NCH
NB = 4
NT = SEQ_PER_W // NB
ROWS = MAXLEN * EMBED // 128


def _body(x_hbm, tok_hbm, pos_hbm, out_hbm, idx_v, pos_v, gbuf, obuf,
          gsems, osems):
    wid = lax.axis_index("s") * NC + lax.axis_index("c")

    pltpu.sync_copy(x_hbm.at[wid], idx_v)
    pltpu.sync_copy(pos_hbm, pos_v)

    def issue_gather(k, b):
        for j in range(NCH):
            pltpu.async_copy(
                tok_hbm.at[idx_v.at[k].at[j]], gbuf.at[b].at[j], gsems[b]
            )

    def wait_gather(k, b):
        for j in range(NCH):
            pltpu.make_async_copy(
                tok_hbm.at[idx_v.at[k].at[j]], gbuf.at[b].at[j], gsems[b]
            ).wait()

    def out_slice(k):
        return out_hbm.at[pl.ds((wid * SEQ_PER_W + k) * ROWS, ROWS)]

    for b in range(NB):
        issue_gather(b, b)

    def outer(t, carry):
        for b in range(NB):
            k = t * NB + b
            wait_gather(k, b)

            @pl.when(t > 0)
            def _():
                pltpu.make_async_copy(obuf.at[b], out_slice(k - NB),
                                      osems[b]).wait()

            def add_body(q, c):
                for j in range(NCH):
                    for rr in range(4):
                        for h in range(EMBED // L):
                            o = pl.ds((rr * EMBED + h * L) % 128, L)
                            g = pl.ds(h * L, L)
                            obuf[b, j * (CH // 4) + q, o] = (
                                gbuf[b, j, q * 4 + rr, g]
                                + pos_v[j * (CH // 4) + q, o]
                            )
                return c

            lax.fori_loop(0, CH // 4, add_body, 0)
            pltpu.async_copy(obuf.at[b], out_slice(k), osems[b])

            @pl.when(t < NT - 1)
            def _():
                issue_gather(k + NB, b)
        return carry

    lax.fori_loop(0, NT, outer, 0)

    for b in range(NB):
        pltpu.make_async_copy(obuf.at[b], out_slice((NT - 1) * NB + b),
                              osems[b]).wait()


@jax.jit
def kernel(x, token_table, pos_table):
    x4 = x.reshape(NW, SEQ_PER_W, NCH, CH).astype(jnp.int32)
    pos2 = pos_table.reshape(ROWS, 128)
    mesh = plsc.VectorSubcoreMesh(
        core_axis_name="c", subcore_axis_name="s", num_cores=NC, num_subcores=NS
    )
    run = pl.kernel(
        _body,
        out_type=jax.ShapeDtypeStruct((BATCH * ROWS, 128), jnp.float32),
        mesh=mesh,
        scratch_types=[
            pltpu.VMEM((SEQ_PER_W, NCH, CH), jnp.int32),
            pltpu.VMEM((ROWS, 128), jnp.float32),
            pltpu.VMEM((NB, NCH, CH, EMBED), jnp.float32),
            pltpu.VMEM((NB, ROWS, 128), jnp.float32),
            [pltpu.SemaphoreType.DMA] * NB,
            [pltpu.SemaphoreType.DMA] * NB,
        ],
        compiler_params=pltpu.CompilerParams(use_tc_tiling_on_sc=False),
    )
    out = run(x4, token_table, pos2)
    return out.reshape(BATCH, MAXLEN, EMBED)
